# baseline (speedup 1.0000x reference)
import functools

import jax
import jax.numpy as jnp
from jax import lax
from jax.experimental import pallas as pl
from jax.experimental.pallas import tpu as pltpu
from jax.experimental.pallas import tpu_sc as plsc

N_FIELDS = 26
VOCAB = 100000
EMB = 16
EPS = 1e-5

NW = 32
CHUNK = 128
GEX = 64
GCH = GEX * N_FIELDS // CHUNK
OUTW = 512


def _gather_kernel(B):
    epw = B // NW
    ngr = epw // GEX
    nch = epw * N_FIELDS // CHUNK
    mesh = plsc.VectorSubcoreMesh(core_axis_name="c", subcore_axis_name="s")

    @functools.partial(
        pl.kernel,
        out_type=jax.ShapeDtypeStruct((B, OUTW), jnp.float32),
        mesh=mesh,
        compiler_params=pltpu.CompilerParams(use_tc_tiling_on_sc=True,
                                             needs_layout_passes=False),
        scratch_types=[
            pltpu.VMEM((nch, CHUNK), jnp.int32),
            pltpu.VMEM((nch, CHUNK), jnp.int32),
            pltpu.VMEM((CHUNK, 128), jnp.float32),
            pltpu.VMEM((GEX, OUTW), jnp.float32),
            pltpu.SemaphoreType.DMA,
        ],
    )
    def gk(t128_hbm, hi_hbm, full_hbm, out_hbm, hi_v, full_v, g_v, out_v, gsem):
        wid = lax.axis_index("s") * 2 + lax.axis_index("c")
        pltpu.sync_copy(hi_hbm.at[wid], hi_v)
        pltpu.sync_copy(full_hbm.at[wid], full_v)

        zeros = jnp.zeros((16,), jnp.float32)
        for e in range(GEX):
            for k in range((OUTW - EMB * N_FIELDS) // 16):
                out_v[e, pl.ds(EMB * N_FIELDS + 16 * k, 16)] = zeros

        iota = lax.iota(jnp.int32, 16)

        def group_body(g, carry):
            def chunk_body(cc, carry2):
                c = g * GCH + cc
                pltpu.async_copy(t128_hbm.at[hi_v.at[c]], g_v, gsem).wait()
                for j0 in range(CHUNK // 16):
                    fl = full_v[c, pl.ds(j0 * 16, 16)]
                    s16 = (fl & 7) * 16
                    rg = cc * CHUNK + j0 * 16 + iota
                    e_rel = (rg * 2521) >> 16
                    fcol = (rg - e_rel * N_FIELDS) * 16
                    row = j0 * 16 + iota
                    for l in range(EMB):
                        v = plsc.load_gather(g_v, [row, s16 + l])
                        plsc.store_scatter(out_v, [e_rel, fcol + l], v)
                return carry2

            lax.fori_loop(0, GCH, chunk_body, 0)
            pltpu.sync_copy(out_v, out_hbm.at[pl.ds(wid * epw + g * GEX, GEX)])
            return carry

        lax.fori_loop(0, ngr, group_body, 0)

    return gk


def _mlp_body(xn_ref, emb_ref, w0n_ref, w0e_ref, b0_ref, w1_ref, b1_ref,
              w2_ref, b2_ref, w3_ref, b3_ref, out_ref):
    h = jnp.dot(emb_ref[...], w0e_ref[...], preferred_element_type=jnp.float32)
    h = h + jnp.dot(xn_ref[...], w0n_ref[...], preferred_element_type=jnp.float32)
    h = jnp.maximum(h + b0_ref[...], 0.0)
    h = jnp.maximum(jnp.dot(h, w1_ref[...], preferred_element_type=jnp.float32)
                    + b1_ref[...], 0.0)
    h = jnp.maximum(jnp.dot(h, w2_ref[...], preferred_element_type=jnp.float32)
                    + b2_ref[...], 0.0)
    out_ref[...] = jnp.dot(h, w3_ref[...], preferred_element_type=jnp.float32) + b3_ref[...]


def kernel(x_num, x_cat, tables, W0, b0, g0, be0, W1, b1, g1, be1,
           W2, b2, g2, be2, W3, b3):
    B = x_num.shape[0]
    epw = B // NW
    nch = epw * N_FIELDS // CHUNK

    t128 = tables.reshape(N_FIELDS * VOCAB // 8, 128)
    offs = (jnp.arange(N_FIELDS, dtype=jnp.int32) * VOCAB)[None, :]
    flat = (x_cat.astype(jnp.int32) + offs).reshape(NW, nch, CHUNK)
    hi = flat >> 3

    inv = 1.0 / jnp.sqrt(1.0 + EPS)
    s0, s1, s2 = g0 * inv, g1 * inv, g2 * inv
    W0f = W0 * s0[None, :]
    b0f = (b0 * s0 + be0)[None, :]
    W1f = W1 * s1[None, :]
    b1f = (b1 * s1 + be1)[None, :]
    W2f = W2 * s2[None, :]
    b2f = (b2 * s2 + be2)[None, :]
    W0n = jnp.pad(W0f[:13], ((0, 3), (0, 0)))
    W0e = jnp.pad(W0f[13:], ((0, OUTW - 416), (0, 0)))
    w3p = jnp.pad(W3, ((0, 0), (0, 127)))
    b3p = jnp.pad(b3.reshape(1, 1), ((0, 0), (0, 127)))
    xn = jnp.pad(x_num, ((0, 0), (0, 3)))

    emb2 = _gather_kernel(B)(t128, hi, flat)

    BM = 2048
    nb = B // BM
    full = lambda s: pl.BlockSpec(s, lambda i: (0, 0))
    out2 = pl.pallas_call(
        _mlp_body,
        grid=(nb,),
        in_specs=[
            pl.BlockSpec((BM, 16), lambda i: (i, 0)),
            pl.BlockSpec((BM, OUTW), lambda i: (i, 0)),
            full((16, 128)), full((OUTW, 128)), full((1, 128)),
            full((128, 64)), full((1, 64)),
            full((64, 32)), full((1, 32)),
            full((32, 128)), full((1, 128)),
        ],
        out_specs=pl.BlockSpec((BM, 128), lambda i: (i, 0)),
        out_shape=jax.ShapeDtypeStruct((B, 128), jnp.float32),
    )(xn, emb2, W0n, W0e, b0f, W1f, b1f, W2f, b2f, w3p, b3p)

    return out2[:, 0]


# speedup vs baseline: 1.1661x; 1.1661x over previous
import functools

import jax
import jax.numpy as jnp
from jax import lax
from jax.experimental import pallas as pl
from jax.experimental.pallas import tpu as pltpu
from jax.experimental.pallas import tpu_sc as plsc

N_FIELDS = 26
VOCAB = 100000
EMB = 16
EPS = 1e-5

NW = 32
CHUNK = 128


def _gather_kernel(B):
    epw = B // NW
    ncc = epw // CHUNK
    mesh = plsc.VectorSubcoreMesh(core_axis_name="c", subcore_axis_name="s")

    @functools.partial(
        pl.kernel,
        out_type=jax.ShapeDtypeStruct((B, EMB * N_FIELDS), jnp.float32),
        mesh=mesh,
        compiler_params=pltpu.CompilerParams(use_tc_tiling_on_sc=False),
        scratch_types=[
            pltpu.VMEM((N_FIELDS, ncc, CHUNK), jnp.int32),
            pltpu.VMEM((2, epw, EMB), jnp.float32),
            pltpu.SemaphoreType.DMA,
            pltpu.SemaphoreType.DMA,
            pltpu.SemaphoreType.DMA,
        ],
    )
    def gk(tab_hbm, idx_hbm, out_hbm, idx_v, rows_v, gsem, wsem0, wsem1):
        wid = lax.axis_index("s") * 2 + lax.axis_index("c")
        b0 = wid * epw
        pltpu.sync_copy(idx_hbm.at[wid], idx_v)

        wsems = (wsem0, wsem1)
        writes = [None, None]
        for f in range(N_FIELDS):
            buf = f % 2
            gathers = [
                pltpu.async_copy(
                    tab_hbm.at[f].at[idx_v.at[f, cc]],
                    rows_v.at[buf, pl.ds(cc * CHUNK, CHUNK)],
                    gsem,
                )
                for cc in range(ncc)
            ]
            if writes[buf] is not None:
                writes[buf].wait()
            for g in gathers:
                g.wait()
            writes[buf] = pltpu.async_copy(
                rows_v.at[buf],
                out_hbm.at[pl.ds(b0, epw), pl.ds(EMB * f, EMB)],
                wsems[buf],
            )
        for w in writes:
            if w is not None:
                w.wait()

    return gk


def _mlp_body(xn_ref, emb_ref, w0n_ref, w0e_ref, b0_ref, w1_ref, b1_ref,
              w2_ref, b2_ref, w3_ref, b3_ref, out_ref):
    h = jnp.dot(emb_ref[...], w0e_ref[...], preferred_element_type=jnp.float32)
    h = h + jnp.dot(xn_ref[...], w0n_ref[...], preferred_element_type=jnp.float32)
    h = jnp.maximum(h + b0_ref[...], 0.0)
    h = jnp.maximum(jnp.dot(h, w1_ref[...], preferred_element_type=jnp.float32)
                    + b1_ref[...], 0.0)
    h = jnp.maximum(jnp.dot(h, w2_ref[...], preferred_element_type=jnp.float32)
                    + b2_ref[...], 0.0)
    out_ref[...] = jnp.dot(h, w3_ref[...], preferred_element_type=jnp.float32) + b3_ref[...]


def kernel(x_num, x_cat, tables, W0, b0, g0, be0, W1, b1, g1, be1,
           W2, b2, g2, be2, W3, b3):
    B = x_num.shape[0]
    epw = B // NW
    ncc = epw // CHUNK

    idx = (x_cat.astype(jnp.int32)
           .reshape(NW, epw, N_FIELDS)
           .transpose(0, 2, 1)
           .reshape(NW, N_FIELDS, ncc, CHUNK))

    inv = 1.0 / jnp.sqrt(1.0 + EPS)
    s0, s1, s2 = g0 * inv, g1 * inv, g2 * inv
    W0f = W0 * s0[None, :]
    b0f = (b0 * s0 + be0)[None, :]
    W1f = W1 * s1[None, :]
    b1f = (b1 * s1 + be1)[None, :]
    W2f = W2 * s2[None, :]
    b2f = (b2 * s2 + be2)[None, :]
    W0n = jnp.pad(W0f[:13], ((0, 3), (0, 0)))
    W0e = W0f[13:]
    w3p = jnp.pad(W3, ((0, 0), (0, 127)))
    b3p = jnp.pad(b3.reshape(1, 1), ((0, 0), (0, 127)))
    xn = jnp.pad(x_num, ((0, 0), (0, 3)))

    emb2 = _gather_kernel(B)(tables, idx)

    BM = 2048
    nb = B // BM
    full = lambda s: pl.BlockSpec(s, lambda i: (0, 0))
    out2 = pl.pallas_call(
        _mlp_body,
        grid=(nb,),
        in_specs=[
            pl.BlockSpec((BM, 16), lambda i: (i, 0)),
            pl.BlockSpec((BM, EMB * N_FIELDS), lambda i: (i, 0)),
            full((16, 128)), full((416, 128)), full((1, 128)),
            full((128, 64)), full((1, 64)),
            full((64, 32)), full((1, 32)),
            full((32, 128)), full((1, 128)),
        ],
        out_specs=pl.BlockSpec((BM, 128), lambda i: (i, 0)),
        out_shape=jax.ShapeDtypeStruct((B, 128), jnp.float32),
    )(xn, emb2, W0n, W0e, b0f, W1f, b1f, W2f, b2f, w3p, b3p)

    return out2[:, 0]


# speedup vs baseline: 1.2008x; 1.0297x over previous
import functools

import jax
import jax.numpy as jnp
from jax import lax
from jax.experimental import pallas as pl
from jax.experimental.pallas import tpu as pltpu
from jax.experimental.pallas import tpu_sc as plsc

N_FIELDS = 26
VOCAB = 100000
EMB = 16
EPS = 1e-5

NW = 32
CHUNK = 128
NSTRIPE = 4


def _gather_kernel(B):
    epw = B // NW
    ncc = epw // CHUNK
    mesh = plsc.VectorSubcoreMesh(core_axis_name="c", subcore_axis_name="s")

    @functools.partial(
        pl.kernel,
        out_type=[jax.ShapeDtypeStruct((B, 128), jnp.float32)
                  for _ in range(NSTRIPE)],
        mesh=mesh,
        compiler_params=pltpu.CompilerParams(use_tc_tiling_on_sc=False),
        scratch_types=[
            pltpu.VMEM((N_FIELDS * ncc, CHUNK), jnp.int32),
            pltpu.VMEM((2, epw, EMB), jnp.float32),
            pltpu.SemaphoreType.DMA,
            pltpu.SemaphoreType.DMA,
            pltpu.SemaphoreType.DMA,
        ],
    )
    def gk(tab_hbm, idx_hbm, o0, o1, o2, o3, idx_v, rows_v, gsem, wsem0, wsem1):
        outs = (o0, o1, o2, o3)
        wid = lax.axis_index("s") * 2 + lax.axis_index("c")
        b0 = wid * epw
        pltpu.sync_copy(idx_hbm.at[wid], idx_v)

        wsems = (wsem0, wsem1)
        writes = [None, None]
        for f in range(N_FIELDS):
            buf = f % 2
            gathers = [
                pltpu.async_copy(
                    tab_hbm.at[f].at[idx_v.at[f * ncc + cc]],
                    rows_v.at[buf, pl.ds(cc * CHUNK, CHUNK)],
                    gsem,
                )
                for cc in range(ncc)
            ]
            if writes[buf] is not None:
                writes[buf].wait()
            for g in gathers:
                g.wait()
            writes[buf] = pltpu.async_copy(
                rows_v.at[buf],
                outs[f // 8].at[pl.ds(b0, epw), pl.ds(EMB * (f % 8), EMB)],
                wsems[buf],
            )
        for w in writes:
            if w is not None:
                w.wait()

    return gk


def _mlp_body(xn_ref, e0_ref, e1_ref, e2_ref, e3_ref, w0n_ref, w0a_ref,
              w0b_ref, w0c_ref, w0d_ref, b0_ref, w1_ref, b1_ref,
              w2_ref, b2_ref, w3_ref, b3_ref, out_ref):
    h = jnp.dot(e0_ref[...], w0a_ref[...], preferred_element_type=jnp.float32)
    h = h + jnp.dot(e1_ref[...], w0b_ref[...], preferred_element_type=jnp.float32)
    h = h + jnp.dot(e2_ref[...], w0c_ref[...], preferred_element_type=jnp.float32)
    h = h + jnp.dot(e3_ref[...][:, :32], w0d_ref[...],
                    preferred_element_type=jnp.float32)
    h = h + jnp.dot(xn_ref[...], w0n_ref[...], preferred_element_type=jnp.float32)
    h = jnp.maximum(h + b0_ref[...], 0.0)
    h = jnp.maximum(jnp.dot(h, w1_ref[...], preferred_element_type=jnp.float32)
                    + b1_ref[...], 0.0)
    h = jnp.maximum(jnp.dot(h, w2_ref[...], preferred_element_type=jnp.float32)
                    + b2_ref[...], 0.0)
    out_ref[...] = jnp.dot(h, w3_ref[...], preferred_element_type=jnp.float32) + b3_ref[...]


def kernel(x_num, x_cat, tables, W0, b0, g0, be0, W1, b1, g1, be1,
           W2, b2, g2, be2, W3, b3):
    B = x_num.shape[0]
    epw = B // NW
    ncc = epw // CHUNK

    idx = (x_cat.astype(jnp.int32)
           .reshape(NW, epw, N_FIELDS)
           .transpose(0, 2, 1)
           .reshape(NW, N_FIELDS * ncc, CHUNK))

    inv = 1.0 / jnp.sqrt(1.0 + EPS)
    s0, s1, s2 = g0 * inv, g1 * inv, g2 * inv
    W0f = W0 * s0[None, :]
    b0f = (b0 * s0 + be0)[None, :]
    W1f = W1 * s1[None, :]
    b1f = (b1 * s1 + be1)[None, :]
    W2f = W2 * s2[None, :]
    b2f = (b2 * s2 + be2)[None, :]
    W0n = jnp.pad(W0f[:13], ((0, 3), (0, 0)))
    W0e = W0f[13:]
    W0a, W0b, W0c, W0d = (W0e[:128], W0e[128:256], W0e[256:384], W0e[384:])
    w3p = jnp.pad(W3, ((0, 0), (0, 127)))
    b3p = jnp.pad(b3.reshape(1, 1), ((0, 0), (0, 127)))
    xn = jnp.pad(x_num, ((0, 0), (0, 3)))

    e0, e1, e2, e3 = _gather_kernel(B)(tables, idx)

    BM = 2048
    nb = B // BM
    blk = lambda s: pl.BlockSpec(s, lambda i: (i, 0))
    full = lambda s: pl.BlockSpec(s, lambda i: (0, 0))
    out2 = pl.pallas_call(
        _mlp_body,
        grid=(nb,),
        in_specs=[
            blk((BM, 16)),
            blk((BM, 128)), blk((BM, 128)), blk((BM, 128)), blk((BM, 128)),
            full((16, 128)), full((128, 128)), full((128, 128)),
            full((128, 128)), full((32, 128)), full((1, 128)),
            full((128, 64)), full((1, 64)),
            full((64, 32)), full((1, 32)),
            full((32, 128)), full((1, 128)),
        ],
        out_specs=pl.BlockSpec((BM, 128), lambda i: (i, 0)),
        out_shape=jax.ShapeDtypeStruct((B, 128), jnp.float32),
    )(xn, e0, e1, e2, e3, W0n, W0a, W0b, W0c, W0d, b0f,
      W1f, b1f, W2f, b2f, w3p, b3p)

    return out2[:, 0]
